# Initial kernel scaffold; baseline (speedup 1.0000x reference)
import jax
import jax.numpy as jnp
from jax.experimental import pallas as pl


def kernel(x, edge_index, edge_attr, batch, degree, W_en, b_en, W_ne, b_ne, W_ag, b_ag, W_m0, b_m0, W_u0, b_u0, W_m1, b_m1, W_u1, b_u1, W_m2, b_m2, W_u2, b_u2, W_gg, b_gg, W_ro, b_ro):
    raise NotImplementedError("write your pallas kernel here")


# speedup vs baseline: 3.5281x; 3.5281x over previous
import functools

import jax
import jax.numpy as jnp
from jax import lax
from jax.experimental import pallas as pl
from jax.experimental.pallas import tpu as pltpu
from jax.experimental.pallas import tpu_sc as plsc

NN = 10000
EE = 320000
DD = 128
DH = 64
NC = 2
NS = 16
NP = 10240
RB = 2048
GRID = NP // RB
KCH = 128
EP = 327680
EPT = EP // NS
NCHUNK = EPT // KCH
STRIPE = NP // NS
PAD_BATCH = 17

_f32 = jnp.float32
_HI = lax.Precision.HIGHEST


def _zero_rows(rows):
    def body(i, _):
        for k in range(DH // 16):
            rows[i, pl.ds(16 * k, 16)] = jnp.zeros((16,), _f32)
        return 0
    lax.fori_loop(0, KCH, body, 0)


def _zero_acc(s, rows, acc):
    for k in range(STRIPE // KCH):
        pltpu.sync_copy(rows, acc.at[pl.ds(s * STRIPE + k * KCH, KCH)])


def _adjust_col(col2, c):
    def body(g, _):
        for k in range(KCH // 16):
            v = col2[g, pl.ds(16 * k, 16)]
            col2[g, pl.ds(16 * k, 16)] = v * 2 + c
        return 0
    lax.fori_loop(0, NCHUNK, body, 0)


def _sc_pass0_body(xs_hbm, col_hbm, row_hbm, ea_hbm, sum_out, cnt_out, ea_out,
                   col2, row2, ea2, onesv, rows0, rows1, acc, acc_c, acc_e,
                   gs0, gs1, ss0, ss1):
    c = lax.axis_index("c")
    s = lax.axis_index("s")
    _zero_rows(rows0)
    for k in range(KCH // 16):
        onesv[pl.ds(16 * k, 16)] = jnp.ones((16,), _f32)
    _zero_acc(s, rows0, acc)
    for k in range(STRIPE // DH):
        pltpu.sync_copy(rows0.at[0], acc_c.at[pl.ds(s * STRIPE + k * DH, DH)])
        pltpu.sync_copy(rows0.at[0], acc_e.at[pl.ds(s * STRIPE + k * DH, DH)])
    pltpu.sync_copy(col_hbm.at[s], col2)
    pltpu.sync_copy(row_hbm.at[s], row2)
    pltpu.sync_copy(ea_hbm.at[s], ea2)
    _adjust_col(col2, c)
    plsc.subcore_barrier()

    pltpu.async_copy(xs_hbm.at[col2.at[0]], rows0, gs0)
    pltpu.async_copy(xs_hbm.at[col2.at[1]], rows1, gs1)

    def half(g, rows, gsem, ssem):
        pltpu.make_async_copy(xs_hbm.at[col2.at[g]], rows, gsem).wait()
        pltpu.async_copy(rows, acc.at[row2.at[g]], ssem, add=True)

        @pl.when(c == 0)
        def _():
            pltpu.async_copy(onesv, acc_c.at[row2.at[g]], ssem, add=True)
            pltpu.async_copy(ea2.at[g], acc_e.at[row2.at[g]], ssem, add=True)

        pltpu.make_async_copy(rows, acc.at[row2.at[g]], ssem).wait()

        @pl.when(c == 0)
        def _():
            pltpu.make_async_copy(onesv, acc_c.at[row2.at[g]], ssem).wait()
            pltpu.make_async_copy(ea2.at[g], acc_e.at[row2.at[g]], ssem).wait()

        gn = jnp.minimum(g + 2, NCHUNK - 1)

        @pl.when(g + 2 < NCHUNK)
        def _():
            pltpu.async_copy(xs_hbm.at[col2.at[gn]], rows, gsem)

    def pair(h, _):
        half(2 * h, rows0, gs0, ss0)
        half(2 * h + 1, rows1, gs1, ss1)
        return 0

    lax.fori_loop(0, NCHUNK // 2, pair, 0)
    plsc.subcore_barrier()
    pltpu.sync_copy(acc.at[pl.ds(s * STRIPE, STRIPE)],
                    sum_out.at[c, pl.ds(s * STRIPE, STRIPE)])

    @pl.when(c == 0)
    def _():
        pltpu.sync_copy(acc_c.at[pl.ds(s * STRIPE, STRIPE)],
                        cnt_out.at[pl.ds(s * STRIPE, STRIPE)])
        pltpu.sync_copy(acc_e.at[pl.ds(s * STRIPE, STRIPE)],
                        ea_out.at[pl.ds(s * STRIPE, STRIPE)])


def _sc_wpass_body(xs_hbm, col_hbm, row_hbm, ea_hbm, sum_out,
                   col2, row2, ea2, rows0, rows1, acc,
                   gs0, gs1, ss0, ss1):
    c = lax.axis_index("c")
    s = lax.axis_index("s")
    _zero_rows(rows0)
    _zero_acc(s, rows0, acc)
    pltpu.sync_copy(col_hbm.at[s], col2)
    pltpu.sync_copy(row_hbm.at[s], row2)
    pltpu.sync_copy(ea_hbm.at[s], ea2)
    _adjust_col(col2, c)
    plsc.subcore_barrier()

    pltpu.async_copy(xs_hbm.at[col2.at[0]], rows0, gs0)
    pltpu.async_copy(xs_hbm.at[col2.at[1]], rows1, gs1)

    def half(g, rows, gsem, ssem):
        pltpu.make_async_copy(xs_hbm.at[col2.at[g]], rows, gsem).wait()

        def scale(gg, _):
            ea16 = ea2[g, pl.ds(gg * 16, 16)]
            for j2 in range(16):
                sc_ = jnp.broadcast_to(ea16[j2], (16,))
                idx = gg * 16 + j2
                for k in range(DH // 16):
                    rows[idx, pl.ds(16 * k, 16)] = (
                        rows[idx, pl.ds(16 * k, 16)] * sc_)
            return 0

        lax.fori_loop(0, KCH // 16, scale, 0)
        pltpu.async_copy(rows, acc.at[row2.at[g]], ssem, add=True)
        pltpu.make_async_copy(rows, acc.at[row2.at[g]], ssem).wait()
        gn = jnp.minimum(g + 2, NCHUNK - 1)

        @pl.when(g + 2 < NCHUNK)
        def _():
            pltpu.async_copy(xs_hbm.at[col2.at[gn]], rows, gsem)

    def pair(h, _):
        half(2 * h, rows0, gs0, ss0)
        half(2 * h + 1, rows1, gs1, ss1)
        return 0

    lax.fori_loop(0, NCHUNK // 2, pair, 0)
    plsc.subcore_barrier()
    pltpu.sync_copy(acc.at[pl.ds(s * STRIPE, STRIPE)],
                    sum_out.at[c, pl.ds(s * STRIPE, STRIPE)])


@functools.lru_cache(maxsize=None)
def _sc_kernels():
    mesh = plsc.VectorSubcoreMesh(core_axis_name="c", subcore_axis_name="s",
                                  num_cores=NC, num_subcores=NS)
    cp = pltpu.CompilerParams(use_tc_tiling_on_sc=False)
    pass0 = pl.kernel(
        _sc_pass0_body,
        compiler_params=cp,
        out_type=(
            jax.ShapeDtypeStruct((NC, NP, DH), _f32),
            jax.ShapeDtypeStruct((NP,), _f32),
            jax.ShapeDtypeStruct((NP,), _f32),
        ),
        mesh=mesh,
        scratch_types=[
            pltpu.VMEM((NCHUNK, KCH), jnp.int32),
            pltpu.VMEM((NCHUNK, KCH), jnp.int32),
            pltpu.VMEM((NCHUNK, KCH), _f32),
            pltpu.VMEM((KCH,), _f32),
            pltpu.VMEM((KCH, DH), _f32),
            pltpu.VMEM((KCH, DH), _f32),
            pltpu.VMEM_SHARED((NP, DH), _f32),
            pltpu.VMEM_SHARED((NP,), _f32),
            pltpu.VMEM_SHARED((NP,), _f32),
            pltpu.SemaphoreType.DMA,
            pltpu.SemaphoreType.DMA,
            pltpu.SemaphoreType.DMA,
            pltpu.SemaphoreType.DMA,
        ],
    )
    wpass = pl.kernel(
        _sc_wpass_body,
        compiler_params=cp,
        out_type=jax.ShapeDtypeStruct((NC, NP, DH), _f32),
        mesh=mesh,
        scratch_types=[
            pltpu.VMEM((NCHUNK, KCH), jnp.int32),
            pltpu.VMEM((NCHUNK, KCH), jnp.int32),
            pltpu.VMEM((NCHUNK, KCH), _f32),
            pltpu.VMEM((KCH, DH), _f32),
            pltpu.VMEM((KCH, DH), _f32),
            pltpu.VMEM_SHARED((NP, DH), _f32),
            pltpu.SemaphoreType.DMA,
            pltpu.SemaphoreType.DMA,
            pltpu.SemaphoreType.DMA,
            pltpu.SemaphoreType.DMA,
        ],
    )
    return pass0, wpass


def _iota_lanes():
    return lax.broadcasted_iota(jnp.int32, (1, DD), 1)


def _tc_embed_body(x_ref, w_ref, b_ref, o_ref):
    o_ref[...] = jnp.maximum(
        jnp.dot(x_ref[...], w_ref[...], preferred_element_type=_f32)
        + b_ref[...], 0.0)


def _tc_deg_body(deg_ref, bat_ref, o_ref):
    i = pl.program_id(0)
    oh = bat_ref[...] == _iota_lanes()
    part = jnp.max(jnp.where(oh, deg_ref[...], -1e30), axis=0, keepdims=True)
    part8 = jnp.broadcast_to(part, (8, DD))

    @pl.when(i == 0)
    def _():
        o_ref[...] = part8

    @pl.when(i > 0)
    def _():
        o_ref[...] = jnp.maximum(o_ref[...], part8)


def _tc_stage2_body(sum_ref, cnt_ref, eas_ref, bat_ref, dm_ref,
                    wne_ref, bne_ref, wag_ref, bag_ref, o_ref):
    rec = 1.0 / jnp.maximum(cnt_ref[...], 1.0)
    sx = jnp.concatenate([sum_ref[0], sum_ref[1]], axis=1) * rec
    eam = eas_ref[...] * rec
    nein = jnp.concatenate([sx, eam], axis=1)
    ne = jnp.maximum(
        jnp.dot(nein, wne_ref[...], preferred_element_type=_f32)
        + bne_ref[...], 0.0)
    oh = (bat_ref[...] == _iota_lanes()).astype(_f32)
    dn = jnp.sum(oh * dm_ref[0:1, :], axis=1, keepdims=True)
    aggin = jnp.concatenate([ne, dn], axis=1)
    o_ref[...] = jnp.maximum(
        jnp.dot(aggin, wag_ref[...], preferred_element_type=_f32)
        + bag_ref[...], 0.0)


def _tc_layer_body(sum_ref, cnt_ref, xe_ref, xae_ref,
                   wm_ref, bm_ref, wu_ref, bu_ref, o_ref):
    rec = 1.0 / jnp.maximum(cnt_ref[...], 1.0)
    xa = jnp.concatenate([sum_ref[0], sum_ref[1]], axis=1) * rec
    m = jnp.maximum(
        jnp.dot(jnp.concatenate([xa, xae_ref[...]], axis=1), wm_ref[...],
                preferred_element_type=_f32)
        + bm_ref[...], 0.0)
    o_ref[...] = jnp.maximum(
        jnp.dot(jnp.concatenate([xe_ref[...], m], axis=1), wu_ref[...],
                preferred_element_type=_f32)
        + bu_ref[...], 0.0)


def _tc_pool_body(xe_ref, bat_ref, gs_ref, nb_ref):
    i = pl.program_id(0)
    ohf = (bat_ref[...] == _iota_lanes()).astype(_f32)
    gsp = lax.dot_general(ohf, xe_ref[...], (((0,), (0,)), ((), ())),
                          preferred_element_type=_f32,
                          precision=_HI)
    nbp = jnp.broadcast_to(jnp.sum(ohf, axis=0, keepdims=True), (8, DD))

    @pl.when(i == 0)
    def _():
        gs_ref[...] = gsp
        nb_ref[...] = nbp

    @pl.when(i > 0)
    def _():
        gs_ref[...] = gs_ref[...] + gsp
        nb_ref[...] = nb_ref[...] + nbp


def _tc_final_body(gs_ref, nb_ref, bat_ref, xe_ref,
                   wgg_ref, bgg_ref, wro_ref, bro_ref, o_ref):
    ohf = (bat_ref[...] == _iota_lanes()).astype(_f32)
    gbs = jnp.dot(ohf, gs_ref[...], preferred_element_type=_f32,
                  precision=_HI)
    nbn = jnp.maximum(
        jnp.sum(ohf * nb_ref[0:1, :], axis=1, keepdims=True), 1.0)
    gb = (jnp.dot(gbs / nbn, wgg_ref[...], preferred_element_type=_f32)
          + bgg_ref[...])
    inp = jnp.concatenate([jnp.maximum(gb, 0.0), xe_ref[...]], axis=1)
    o_ref[...] = (
        jnp.dot(inp, wro_ref[...], preferred_element_type=_f32)
        + bro_ref[...])


def _rowspec(shape):
    if shape == (NP, DD):
        return pl.BlockSpec((RB, DD), lambda i: (i, 0))
    if shape == (NP, 1):
        return pl.BlockSpec((RB, 1), lambda i: (i, 0))
    if shape == (NC, NP, DH):
        return pl.BlockSpec((NC, RB, DH), lambda i: (0, i, 0))
    return pl.BlockSpec(shape, lambda i: tuple(0 for _ in shape))


def _tc_call(body, args, out_shapes):
    single = not isinstance(out_shapes[0], tuple)
    oss = (out_shapes,) if single else tuple(out_shapes)
    res = pl.pallas_call(
        body,
        grid=(GRID,),
        in_specs=[_rowspec(a.shape) for a in args],
        out_specs=tuple(_rowspec(s) for s in oss) if not single
        else _rowspec(out_shapes),
        out_shape=tuple(jax.ShapeDtypeStruct(s, _f32) for s in oss)
        if not single else jax.ShapeDtypeStruct(out_shapes, _f32),
    )(*args)
    return res


def kernel(x, edge_index, edge_attr, batch, degree,
           W_en, b_en, W_ne, b_ne, W_ag, b_ag,
           W_m0, b_m0, W_u0, b_u0, W_m1, b_m1, W_u1, b_u1,
           W_m2, b_m2, W_u2, b_u2, W_gg, b_gg, W_ro, b_ro):
    x2 = jnp.pad(x.reshape(NN, DD), ((0, NP - NN), (0, 0)))
    col = edge_index[0]
    row = edge_index[1]
    ea = edge_attr[:, 0]
    npad = EP - EE
    pad_rows = NN + (jnp.arange(npad, dtype=jnp.int32) % (NP - NN))
    col_p = jnp.concatenate(
        [col, jnp.zeros((npad,), jnp.int32)]).reshape(NS, NCHUNK, KCH)
    row_p = jnp.concatenate([row, pad_rows]).reshape(NS, NCHUNK, KCH)
    ea_p = jnp.concatenate(
        [ea, jnp.zeros((npad,), _f32)]).reshape(NS, NCHUNK, KCH)
    bat_p = jnp.concatenate(
        [batch, jnp.full((NP - NN,), PAD_BATCH, jnp.int32)]).reshape(NP, 1)
    deg_p = jnp.pad(degree, (0, NP - NN)).reshape(NP, 1)

    b2 = lambda b: b.reshape(1, -1)
    wro = jnp.pad(W_ro, ((0, 0), (0, DD - 1)))
    bro = jnp.pad(b_ro, (0, DD - 1)).reshape(1, DD)

    x_emb = _tc_call(_tc_embed_body, (x2, W_en, b2(b_en)), (NP, DD))
    degmax = _tc_call(_tc_deg_body, (deg_p, bat_p), (8, DD))

    sc_pass0, sc_wpass = _sc_kernels()
    s0, cnt, eas = sc_pass0(x2.reshape(NP * 2, DH), col_p, row_p, ea_p)
    cnt3 = cnt.reshape(NP, 1)
    eas3 = eas.reshape(NP, 1)
    x_emb, _ = lax.optimization_barrier((x_emb, s0))

    x_agg_emb = _tc_call(
        _tc_stage2_body,
        (s0, cnt3, eas3, bat_p, degmax, W_ne, b2(b_ne), W_ag, b2(b_ag)),
        (NP, DD))

    for Wm, bm, Wu, bu in ((W_m0, b_m0, W_u0, b_u0),
                           (W_m1, b_m1, W_u1, b_u1),
                           (W_m2, b_m2, W_u2, b_u2)):
        sl = sc_wpass(x_emb.reshape(NP * 2, DH), col_p, row_p, ea_p)
        x_emb = _tc_call(
            _tc_layer_body,
            (sl, cnt3, x_emb, x_agg_emb, Wm, b2(bm), Wu, b2(bu)), (NP, DD))

    gs, nb = _tc_call(
        _tc_pool_body, (x_emb, bat_p), ((DD, DD), (8, DD)))
    q = _tc_call(
        _tc_final_body,
        (gs, nb, bat_p, x_emb, W_gg, b2(b_gg), wro, bro), (NP, DD))
    return q[:NN, 0:1]


# speedup vs baseline: 3.6271x; 1.0281x over previous
import functools

import jax
import jax.numpy as jnp
from jax import lax
from jax.experimental import pallas as pl
from jax.experimental.pallas import tpu as pltpu
from jax.experimental.pallas import tpu_sc as plsc

NN = 10000
EE = 320000
DD = 128
DH = 64
NC = 2
NS = 16
NP = 10240
RB = 2048
GRID = NP // RB
KCH = 128
EP = 327680
EPT = EP // NS
NCHUNK = EPT // KCH
STRIPE = NP // NS
PAD_BATCH = 17

_f32 = jnp.float32
_HI = lax.Precision.HIGHEST


def _zero_rows(rows):
    def body(i, _):
        for k in range(DH // 16):
            rows[i, pl.ds(16 * k, 16)] = jnp.zeros((16,), _f32)
        return 0
    lax.fori_loop(0, KCH, body, 0)


def _zero_acc(s, rows, acc):
    for k in range(STRIPE // KCH):
        pltpu.sync_copy(rows, acc.at[pl.ds(s * STRIPE + k * KCH, KCH)])


def _adjust_col(col2, c):
    def body(g, _):
        for k in range(KCH // 16):
            v = col2[g, pl.ds(16 * k, 16)]
            col2[g, pl.ds(16 * k, 16)] = v * 2 + c
        return 0
    lax.fori_loop(0, NCHUNK, body, 0)


def _sc_pass0_body(xs_hbm, col_hbm, row_hbm, ea_hbm, sum_out, cnt_out, ea_out,
                   col2, row2, ea2, onesv, rows0, rows1, acc, acc_c, acc_e,
                   gs0, gs1, ss0, ss1):
    c = lax.axis_index("c")
    s = lax.axis_index("s")
    _zero_rows(rows0)
    for k in range(KCH // 16):
        onesv[pl.ds(16 * k, 16)] = jnp.ones((16,), _f32)
    _zero_acc(s, rows0, acc)
    for k in range(STRIPE // DH):
        pltpu.sync_copy(rows0.at[0], acc_c.at[pl.ds(s * STRIPE + k * DH, DH)])
        pltpu.sync_copy(rows0.at[0], acc_e.at[pl.ds(s * STRIPE + k * DH, DH)])
    pltpu.sync_copy(col_hbm.at[s], col2)
    pltpu.sync_copy(row_hbm.at[s], row2)
    pltpu.sync_copy(ea_hbm.at[s], ea2)
    _adjust_col(col2, c)
    plsc.subcore_barrier()

    pltpu.async_copy(xs_hbm.at[col2.at[0]], rows0, gs0)
    pltpu.async_copy(xs_hbm.at[col2.at[1]], rows1, gs1)

    def half(g, rows, gsem, ssem):
        pltpu.make_async_copy(xs_hbm.at[col2.at[g]], rows, gsem).wait()
        pltpu.async_copy(rows, acc.at[row2.at[g]], ssem, add=True)

        @pl.when(c == 0)
        def _():
            pltpu.async_copy(onesv, acc_c.at[row2.at[g]], ssem, add=True)
            pltpu.async_copy(ea2.at[g], acc_e.at[row2.at[g]], ssem, add=True)

        pltpu.make_async_copy(rows, acc.at[row2.at[g]], ssem).wait()

        @pl.when(c == 0)
        def _():
            pltpu.make_async_copy(onesv, acc_c.at[row2.at[g]], ssem).wait()
            pltpu.make_async_copy(ea2.at[g], acc_e.at[row2.at[g]], ssem).wait()

        gn = jnp.minimum(g + 2, NCHUNK - 1)

        @pl.when(g + 2 < NCHUNK)
        def _():
            pltpu.async_copy(xs_hbm.at[col2.at[gn]], rows, gsem)

    def pair(h, _):
        half(2 * h, rows0, gs0, ss0)
        half(2 * h + 1, rows1, gs1, ss1)
        return 0

    lax.fori_loop(0, NCHUNK // 2, pair, 0)
    plsc.subcore_barrier()
    pltpu.sync_copy(acc.at[pl.ds(s * STRIPE, STRIPE)],
                    sum_out.at[c, pl.ds(s * STRIPE, STRIPE)])

    @pl.when(c == 0)
    def _():
        pltpu.sync_copy(acc_c.at[pl.ds(s * STRIPE, STRIPE)],
                        cnt_out.at[pl.ds(s * STRIPE, STRIPE)])
        pltpu.sync_copy(acc_e.at[pl.ds(s * STRIPE, STRIPE)],
                        ea_out.at[pl.ds(s * STRIPE, STRIPE)])


def _sc_wpass_body(xs_hbm, col_hbm, row_hbm, ea_hbm, sum_out,
                   col2, row2, ea2, rows0, rows1, acc,
                   gs0, gs1, ss0, ss1):
    c = lax.axis_index("c")
    s = lax.axis_index("s")
    _zero_rows(rows0)
    _zero_acc(s, rows0, acc)
    pltpu.sync_copy(col_hbm.at[s], col2)
    pltpu.sync_copy(row_hbm.at[s], row2)
    pltpu.sync_copy(ea_hbm.at[s], ea2)
    _adjust_col(col2, c)
    plsc.subcore_barrier()

    pltpu.async_copy(xs_hbm.at[col2.at[0]], rows0, gs0)
    pltpu.async_copy(xs_hbm.at[col2.at[1]], rows1, gs1)

    def half(g, rows, gsem, ssem):
        pltpu.make_async_copy(xs_hbm.at[col2.at[g]], rows, gsem).wait()

        @plsc.parallel_loop(0, KCH // 16, unroll=2)
        def _(gg):
            ea16 = ea2[g, pl.ds(gg * 16, 16)]
            for j2 in range(16):
                sc_ = jnp.broadcast_to(ea16[j2], (16,))
                idx = gg * 16 + j2
                for k in range(DH // 16):
                    rows[idx, pl.ds(16 * k, 16)] = (
                        rows[idx, pl.ds(16 * k, 16)] * sc_)
        pltpu.async_copy(rows, acc.at[row2.at[g]], ssem, add=True)
        pltpu.make_async_copy(rows, acc.at[row2.at[g]], ssem).wait()
        gn = jnp.minimum(g + 2, NCHUNK - 1)

        @pl.when(g + 2 < NCHUNK)
        def _():
            pltpu.async_copy(xs_hbm.at[col2.at[gn]], rows, gsem)

    def pair(h, _):
        half(2 * h, rows0, gs0, ss0)
        half(2 * h + 1, rows1, gs1, ss1)
        return 0

    lax.fori_loop(0, NCHUNK // 2, pair, 0)
    plsc.subcore_barrier()
    pltpu.sync_copy(acc.at[pl.ds(s * STRIPE, STRIPE)],
                    sum_out.at[c, pl.ds(s * STRIPE, STRIPE)])


@functools.lru_cache(maxsize=None)
def _sc_kernels():
    mesh = plsc.VectorSubcoreMesh(core_axis_name="c", subcore_axis_name="s",
                                  num_cores=NC, num_subcores=NS)
    cp = pltpu.CompilerParams(use_tc_tiling_on_sc=False)
    pass0 = pl.kernel(
        _sc_pass0_body,
        compiler_params=cp,
        out_type=(
            jax.ShapeDtypeStruct((NC, NP, DH), _f32),
            jax.ShapeDtypeStruct((NP,), _f32),
            jax.ShapeDtypeStruct((NP,), _f32),
        ),
        mesh=mesh,
        scratch_types=[
            pltpu.VMEM((NCHUNK, KCH), jnp.int32),
            pltpu.VMEM((NCHUNK, KCH), jnp.int32),
            pltpu.VMEM((NCHUNK, KCH), _f32),
            pltpu.VMEM((KCH,), _f32),
            pltpu.VMEM((KCH, DH), _f32),
            pltpu.VMEM((KCH, DH), _f32),
            pltpu.VMEM_SHARED((NP, DH), _f32),
            pltpu.VMEM_SHARED((NP,), _f32),
            pltpu.VMEM_SHARED((NP,), _f32),
            pltpu.SemaphoreType.DMA,
            pltpu.SemaphoreType.DMA,
            pltpu.SemaphoreType.DMA,
            pltpu.SemaphoreType.DMA,
        ],
    )
    wpass = pl.kernel(
        _sc_wpass_body,
        compiler_params=cp,
        out_type=jax.ShapeDtypeStruct((NC, NP, DH), _f32),
        mesh=mesh,
        scratch_types=[
            pltpu.VMEM((NCHUNK, KCH), jnp.int32),
            pltpu.VMEM((NCHUNK, KCH), jnp.int32),
            pltpu.VMEM((NCHUNK, KCH), _f32),
            pltpu.VMEM((KCH, DH), _f32),
            pltpu.VMEM((KCH, DH), _f32),
            pltpu.VMEM_SHARED((NP, DH), _f32),
            pltpu.SemaphoreType.DMA,
            pltpu.SemaphoreType.DMA,
            pltpu.SemaphoreType.DMA,
            pltpu.SemaphoreType.DMA,
        ],
    )
    return pass0, wpass


def _iota_lanes():
    return lax.broadcasted_iota(jnp.int32, (1, DD), 1)


def _tc_embed_body(x_ref, w_ref, b_ref, o_ref):
    o_ref[...] = jnp.maximum(
        jnp.dot(x_ref[...], w_ref[...], preferred_element_type=_f32)
        + b_ref[...], 0.0)


def _tc_deg_body(deg_ref, bat_ref, o_ref):
    i = pl.program_id(0)
    oh = bat_ref[...] == _iota_lanes()
    part = jnp.max(jnp.where(oh, deg_ref[...], -1e30), axis=0, keepdims=True)
    part8 = jnp.broadcast_to(part, (8, DD))

    @pl.when(i == 0)
    def _():
        o_ref[...] = part8

    @pl.when(i > 0)
    def _():
        o_ref[...] = jnp.maximum(o_ref[...], part8)


def _tc_stage2_body(sum_ref, cnt_ref, eas_ref, bat_ref, dm_ref,
                    wne_ref, bne_ref, wag_ref, bag_ref, o_ref):
    rec = 1.0 / jnp.maximum(cnt_ref[...], 1.0)
    sx = jnp.concatenate([sum_ref[0], sum_ref[1]], axis=1) * rec
    eam = eas_ref[...] * rec
    nein = jnp.concatenate([sx, eam], axis=1)
    ne = jnp.maximum(
        jnp.dot(nein, wne_ref[...], preferred_element_type=_f32)
        + bne_ref[...], 0.0)
    oh = (bat_ref[...] == _iota_lanes()).astype(_f32)
    dn = jnp.sum(oh * dm_ref[0:1, :], axis=1, keepdims=True)
    aggin = jnp.concatenate([ne, dn], axis=1)
    o_ref[...] = jnp.maximum(
        jnp.dot(aggin, wag_ref[...], preferred_element_type=_f32)
        + bag_ref[...], 0.0)


def _tc_layer_body(sum_ref, cnt_ref, xe_ref, xae_ref,
                   wm_ref, bm_ref, wu_ref, bu_ref, o_ref):
    rec = 1.0 / jnp.maximum(cnt_ref[...], 1.0)
    xa = jnp.concatenate([sum_ref[0], sum_ref[1]], axis=1) * rec
    m = jnp.maximum(
        jnp.dot(jnp.concatenate([xa, xae_ref[...]], axis=1), wm_ref[...],
                preferred_element_type=_f32)
        + bm_ref[...], 0.0)
    o_ref[...] = jnp.maximum(
        jnp.dot(jnp.concatenate([xe_ref[...], m], axis=1), wu_ref[...],
                preferred_element_type=_f32)
        + bu_ref[...], 0.0)


def _tc_pool_body(xe_ref, bat_ref, gs_ref, nb_ref):
    i = pl.program_id(0)
    ohf = (bat_ref[...] == _iota_lanes()).astype(_f32)
    gsp = lax.dot_general(ohf, xe_ref[...], (((0,), (0,)), ((), ())),
                          preferred_element_type=_f32,
                          precision=_HI)
    nbp = jnp.broadcast_to(jnp.sum(ohf, axis=0, keepdims=True), (8, DD))

    @pl.when(i == 0)
    def _():
        gs_ref[...] = gsp
        nb_ref[...] = nbp

    @pl.when(i > 0)
    def _():
        gs_ref[...] = gs_ref[...] + gsp
        nb_ref[...] = nb_ref[...] + nbp


def _tc_final_body(gs_ref, nb_ref, bat_ref, xe_ref,
                   wgg_ref, bgg_ref, wro_ref, bro_ref, o_ref):
    ohf = (bat_ref[...] == _iota_lanes()).astype(_f32)
    gbs = jnp.dot(ohf, gs_ref[...], preferred_element_type=_f32,
                  precision=_HI)
    nbn = jnp.maximum(
        jnp.sum(ohf * nb_ref[0:1, :], axis=1, keepdims=True), 1.0)
    gb = (jnp.dot(gbs / nbn, wgg_ref[...], preferred_element_type=_f32)
          + bgg_ref[...])
    inp = jnp.concatenate([jnp.maximum(gb, 0.0), xe_ref[...]], axis=1)
    o_ref[...] = (
        jnp.dot(inp, wro_ref[...], preferred_element_type=_f32)
        + bro_ref[...])


def _rowspec(shape):
    if shape == (NP, DD):
        return pl.BlockSpec((RB, DD), lambda i: (i, 0))
    if shape == (NP, 1):
        return pl.BlockSpec((RB, 1), lambda i: (i, 0))
    if shape == (NC, NP, DH):
        return pl.BlockSpec((NC, RB, DH), lambda i: (0, i, 0))
    return pl.BlockSpec(shape, lambda i: tuple(0 for _ in shape))


def _tc_call(body, args, out_shapes):
    single = not isinstance(out_shapes[0], tuple)
    oss = (out_shapes,) if single else tuple(out_shapes)
    res = pl.pallas_call(
        body,
        grid=(GRID,),
        in_specs=[_rowspec(a.shape) for a in args],
        out_specs=tuple(_rowspec(s) for s in oss) if not single
        else _rowspec(out_shapes),
        out_shape=tuple(jax.ShapeDtypeStruct(s, _f32) for s in oss)
        if not single else jax.ShapeDtypeStruct(out_shapes, _f32),
    )(*args)
    return res


def kernel(x, edge_index, edge_attr, batch, degree,
           W_en, b_en, W_ne, b_ne, W_ag, b_ag,
           W_m0, b_m0, W_u0, b_u0, W_m1, b_m1, W_u1, b_u1,
           W_m2, b_m2, W_u2, b_u2, W_gg, b_gg, W_ro, b_ro):
    x2 = jnp.pad(x.reshape(NN, DD), ((0, NP - NN), (0, 0)))
    col = edge_index[0]
    row = edge_index[1]
    ea = edge_attr[:, 0]
    npad = EP - EE
    pad_rows = NN + (jnp.arange(npad, dtype=jnp.int32) % (NP - NN))
    col_p = jnp.concatenate(
        [col, jnp.zeros((npad,), jnp.int32)]).reshape(NS, NCHUNK, KCH)
    row_p = jnp.concatenate([row, pad_rows]).reshape(NS, NCHUNK, KCH)
    ea_p = jnp.concatenate(
        [ea, jnp.zeros((npad,), _f32)]).reshape(NS, NCHUNK, KCH)
    bat_p = jnp.concatenate(
        [batch, jnp.full((NP - NN,), PAD_BATCH, jnp.int32)]).reshape(NP, 1)
    deg_p = jnp.pad(degree, (0, NP - NN)).reshape(NP, 1)

    b2 = lambda b: b.reshape(1, -1)
    wro = jnp.pad(W_ro, ((0, 0), (0, DD - 1)))
    bro = jnp.pad(b_ro, (0, DD - 1)).reshape(1, DD)

    x_emb = _tc_call(_tc_embed_body, (x2, W_en, b2(b_en)), (NP, DD))
    degmax = _tc_call(_tc_deg_body, (deg_p, bat_p), (8, DD))

    sc_pass0, sc_wpass = _sc_kernels()
    s0, cnt, eas = sc_pass0(x2.reshape(NP * 2, DH), col_p, row_p, ea_p)
    cnt3 = cnt.reshape(NP, 1)
    eas3 = eas.reshape(NP, 1)
    x_emb, _ = lax.optimization_barrier((x_emb, s0))

    x_agg_emb = _tc_call(
        _tc_stage2_body,
        (s0, cnt3, eas3, bat_p, degmax, W_ne, b2(b_ne), W_ag, b2(b_ag)),
        (NP, DD))

    for Wm, bm, Wu, bu in ((W_m0, b_m0, W_u0, b_u0),
                           (W_m1, b_m1, W_u1, b_u1),
                           (W_m2, b_m2, W_u2, b_u2)):
        sl = sc_wpass(x_emb.reshape(NP * 2, DH), col_p, row_p, ea_p)
        x_emb = _tc_call(
            _tc_layer_body,
            (sl, cnt3, x_emb, x_agg_emb, Wm, b2(bm), Wu, b2(bu)), (NP, DD))

    gs, nb = _tc_call(
        _tc_pool_body, (x_emb, bat_p), ((DD, DD), (8, DD)))
    q = _tc_call(
        _tc_final_body,
        (gs, nb, bat_p, x_emb, W_gg, b2(b_gg), wro, bro), (NP, DD))
    return q[:NN, 0:1]


# speedup vs baseline: 3.7207x; 1.0258x over previous
import functools

import jax
import jax.numpy as jnp
from jax import lax
from jax.experimental import pallas as pl
from jax.experimental.pallas import tpu as pltpu
from jax.experimental.pallas import tpu_sc as plsc

NN = 10000
EE = 320000
DD = 128
DH = 64
NC = 2
NS = 16
NP = 10240
RB = 2048
GRID = NP // RB
KCH = 128
EP = 327680
EPT = EP // NS
NCHUNK = EPT // KCH
STRIPE = NP // NS
PAD_BATCH = 17

_f32 = jnp.float32
_HI = lax.Precision.HIGHEST


def _zero_rows(rows):
    def body(i, _):
        for k in range(DH // 16):
            rows[i, pl.ds(16 * k, 16)] = jnp.zeros((16,), _f32)
        return 0
    lax.fori_loop(0, KCH, body, 0)


def _zero_acc(s, rows, acc):
    for k in range(STRIPE // KCH):
        pltpu.sync_copy(rows, acc.at[pl.ds(s * STRIPE + k * KCH, KCH)])


def _adjust_col(col2, c):
    def body(g, _):
        for k in range(KCH // 16):
            v = col2[g, pl.ds(16 * k, 16)]
            col2[g, pl.ds(16 * k, 16)] = v * 2 + c
        return 0
    lax.fori_loop(0, NCHUNK, body, 0)


def _issue_gather(xs_hbm, ea_hbm, s, col2, g, rows_b, ea_b, gsem_b):
    pltpu.async_copy(xs_hbm.at[col2.at[g]], rows_b, gsem_b)
    pltpu.async_copy(ea_hbm.at[s, g], ea_b, gsem_b)


def _wait_gather(xs_hbm, ea_hbm, s, col2, g, rows_b, ea_b, gsem_b):
    pltpu.make_async_copy(xs_hbm.at[col2.at[g]], rows_b, gsem_b).wait()
    pltpu.make_async_copy(ea_hbm.at[s, g], ea_b, gsem_b).wait()


def _sc_pass0_body(xs_hbm, col_hbm, row_hbm, ea_hbm, sum_out, cnt_out, ea_out,
                   col2, row2, onesv,
                   rows_a, rows_b, rows_c, rows_d,
                   ea_a, ea_b, ea_c, ea_d, acc, acc_c, acc_e,
                   g0, g1, g2, g3, s0, s1, s2, s3):
    c = lax.axis_index("c")
    s = lax.axis_index("s")
    rows = (rows_a, rows_b, rows_c, rows_d)
    eab = (ea_a, ea_b, ea_c, ea_d)
    gsem = (g0, g1, g2, g3)
    ssem = (s0, s1, s2, s3)
    _zero_rows(rows_a)
    for k in range(KCH // 16):
        onesv[pl.ds(16 * k, 16)] = jnp.ones((16,), _f32)
    _zero_acc(s, rows_a, acc)
    for k in range(STRIPE // DH):
        pltpu.sync_copy(rows_a.at[0], acc_c.at[pl.ds(s * STRIPE + k * DH, DH)])
        pltpu.sync_copy(rows_a.at[0], acc_e.at[pl.ds(s * STRIPE + k * DH, DH)])
    pltpu.sync_copy(col_hbm.at[s], col2)
    pltpu.sync_copy(row_hbm.at[s], row2)
    _adjust_col(col2, c)
    plsc.subcore_barrier()

    for b in range(2):
        _issue_gather(xs_hbm, ea_hbm, s, col2, b, rows[b], eab[b], gsem[b])

    def _drain_scatter(gp, b2):
        pltpu.make_async_copy(rows[b2], acc.at[row2.at[gp]], ssem[b2]).wait()

        @pl.when(c == 0)
        def _():
            pltpu.make_async_copy(onesv, acc_c.at[row2.at[gp]],
                                  ssem[b2]).wait()
            pltpu.make_async_copy(eab[b2], acc_e.at[row2.at[gp]],
                                  ssem[b2]).wait()

    def _step(g, b):
        _wait_gather(xs_hbm, ea_hbm, s, col2, g, rows[b], eab[b], gsem[b])
        pltpu.async_copy(rows[b], acc.at[row2.at[g]], ssem[b], add=True)

        @pl.when(c == 0)
        def _():
            pltpu.async_copy(onesv, acc_c.at[row2.at[g]], ssem[b], add=True)
            pltpu.async_copy(eab[b], acc_e.at[row2.at[g]], ssem[b], add=True)

        b2 = (b + 2) % 4
        gp = jnp.maximum(g - 2, 0)
        gn = jnp.minimum(g + 2, NCHUNK - 1)

        @pl.when(g >= 2)
        def _():
            _drain_scatter(gp, b2)

        @pl.when(g + 2 < NCHUNK)
        def _():
            _issue_gather(xs_hbm, ea_hbm, s, col2, gn, rows[b2], eab[b2],
                          gsem[b2])

    def quad(h, _):
        for b in range(4):
            _step(4 * h + b, b)
        return 0

    lax.fori_loop(0, NCHUNK // 4, quad, 0)
    _drain_scatter(NCHUNK - 2, 2)
    _drain_scatter(NCHUNK - 1, 3)
    plsc.subcore_barrier()
    pltpu.sync_copy(acc.at[pl.ds(s * STRIPE, STRIPE)],
                    sum_out.at[c, pl.ds(s * STRIPE, STRIPE)])

    @pl.when(c == 0)
    def _():
        pltpu.sync_copy(acc_c.at[pl.ds(s * STRIPE, STRIPE)],
                        cnt_out.at[pl.ds(s * STRIPE, STRIPE)])
        pltpu.sync_copy(acc_e.at[pl.ds(s * STRIPE, STRIPE)],
                        ea_out.at[pl.ds(s * STRIPE, STRIPE)])


def _sc_wpass_body(xs_hbm, col_hbm, row_hbm, ea_hbm, sum_out,
                   col2, row2,
                   rows_a, rows_b, rows_c, rows_d,
                   ea_a, ea_b, ea_c, ea_d, acc,
                   g0, g1, g2, g3, s0, s1, s2, s3):
    c = lax.axis_index("c")
    s = lax.axis_index("s")
    rows = (rows_a, rows_b, rows_c, rows_d)
    eab = (ea_a, ea_b, ea_c, ea_d)
    gsem = (g0, g1, g2, g3)
    ssem = (s0, s1, s2, s3)
    _zero_rows(rows_a)
    _zero_acc(s, rows_a, acc)
    pltpu.sync_copy(col_hbm.at[s], col2)
    pltpu.sync_copy(row_hbm.at[s], row2)
    _adjust_col(col2, c)
    plsc.subcore_barrier()

    for b in range(2):
        _issue_gather(xs_hbm, ea_hbm, s, col2, b, rows[b], eab[b], gsem[b])

    def _drain_scatter(gp, b2):
        pltpu.make_async_copy(rows[b2], acc.at[row2.at[gp]], ssem[b2]).wait()

    def _step(g, b):
        _wait_gather(xs_hbm, ea_hbm, s, col2, g, rows[b], eab[b], gsem[b])
        eav = eab[b]
        rv = rows[b]

        @plsc.parallel_loop(0, KCH // 16, unroll=2)
        def _(gg):
            ea16 = eav[pl.ds(gg * 16, 16)]
            for j2 in range(16):
                sc_ = jnp.broadcast_to(ea16[j2], (16,))
                idx = gg * 16 + j2
                for k in range(DH // 16):
                    rv[idx, pl.ds(16 * k, 16)] = (
                        rv[idx, pl.ds(16 * k, 16)] * sc_)

        pltpu.async_copy(rv, acc.at[row2.at[g]], ssem[b], add=True)
        b2 = (b + 2) % 4
        gp = jnp.maximum(g - 2, 0)
        gn = jnp.minimum(g + 2, NCHUNK - 1)

        @pl.when(g >= 2)
        def _():
            _drain_scatter(gp, b2)

        @pl.when(g + 2 < NCHUNK)
        def _():
            _issue_gather(xs_hbm, ea_hbm, s, col2, gn, rows[b2], eab[b2],
                          gsem[b2])

    def quad(h, _):
        for b in range(4):
            _step(4 * h + b, b)
        return 0

    lax.fori_loop(0, NCHUNK // 4, quad, 0)
    _drain_scatter(NCHUNK - 2, 2)
    _drain_scatter(NCHUNK - 1, 3)
    plsc.subcore_barrier()
    pltpu.sync_copy(acc.at[pl.ds(s * STRIPE, STRIPE)],
                    sum_out.at[c, pl.ds(s * STRIPE, STRIPE)])


@functools.lru_cache(maxsize=None)
def _sc_kernels():
    mesh = plsc.VectorSubcoreMesh(core_axis_name="c", subcore_axis_name="s",
                                  num_cores=NC, num_subcores=NS)
    cp = pltpu.CompilerParams(use_tc_tiling_on_sc=False)
    pass0 = pl.kernel(
        _sc_pass0_body,
        compiler_params=cp,
        out_type=(
            jax.ShapeDtypeStruct((NC, NP, DH), _f32),
            jax.ShapeDtypeStruct((NP,), _f32),
            jax.ShapeDtypeStruct((NP,), _f32),
        ),
        mesh=mesh,
        scratch_types=[
            pltpu.VMEM((NCHUNK, KCH), jnp.int32),
            pltpu.VMEM((NCHUNK, KCH), jnp.int32),
            pltpu.VMEM((KCH,), _f32),
        ] + [pltpu.VMEM((KCH, DH), _f32)] * 4
        + [pltpu.VMEM((KCH,), _f32)] * 4
        + [
            pltpu.VMEM_SHARED((NP, DH), _f32),
            pltpu.VMEM_SHARED((NP,), _f32),
            pltpu.VMEM_SHARED((NP,), _f32),
        ] + [pltpu.SemaphoreType.DMA] * 8,
    )
    wpass = pl.kernel(
        _sc_wpass_body,
        compiler_params=cp,
        out_type=jax.ShapeDtypeStruct((NC, NP, DH), _f32),
        mesh=mesh,
        scratch_types=[
            pltpu.VMEM((NCHUNK, KCH), jnp.int32),
            pltpu.VMEM((NCHUNK, KCH), jnp.int32),
        ] + [pltpu.VMEM((KCH, DH), _f32)] * 4
        + [pltpu.VMEM((KCH,), _f32)] * 4
        + [pltpu.VMEM_SHARED((NP, DH), _f32)]
        + [pltpu.SemaphoreType.DMA] * 8,
    )
    return pass0, wpass


def _iota_lanes():
    return lax.broadcasted_iota(jnp.int32, (1, DD), 1)


def _tc_embed_body(x_ref, w_ref, b_ref, o_ref):
    o_ref[...] = jnp.maximum(
        jnp.dot(x_ref[...], w_ref[...], preferred_element_type=_f32)
        + b_ref[...], 0.0)


def _tc_deg_body(deg_ref, bat_ref, o_ref):
    i = pl.program_id(0)
    oh = bat_ref[...] == _iota_lanes()
    part = jnp.max(jnp.where(oh, deg_ref[...], -1e30), axis=0, keepdims=True)
    part8 = jnp.broadcast_to(part, (8, DD))

    @pl.when(i == 0)
    def _():
        o_ref[...] = part8

    @pl.when(i > 0)
    def _():
        o_ref[...] = jnp.maximum(o_ref[...], part8)


def _tc_stage2_body(sum_ref, cnt_ref, eas_ref, bat_ref, dm_ref,
                    wne_ref, bne_ref, wag_ref, bag_ref, o_ref):
    rec = 1.0 / jnp.maximum(cnt_ref[...], 1.0)
    sx = jnp.concatenate([sum_ref[0], sum_ref[1]], axis=1) * rec
    eam = eas_ref[...] * rec
    nein = jnp.concatenate([sx, eam], axis=1)
    ne = jnp.maximum(
        jnp.dot(nein, wne_ref[...], preferred_element_type=_f32)
        + bne_ref[...], 0.0)
    oh = (bat_ref[...] == _iota_lanes()).astype(_f32)
    dn = jnp.sum(oh * dm_ref[0:1, :], axis=1, keepdims=True)
    aggin = jnp.concatenate([ne, dn], axis=1)
    o_ref[...] = jnp.maximum(
        jnp.dot(aggin, wag_ref[...], preferred_element_type=_f32)
        + bag_ref[...], 0.0)


def _tc_layer_body(sum_ref, cnt_ref, xe_ref, xae_ref,
                   wm_ref, bm_ref, wu_ref, bu_ref, o_ref):
    rec = 1.0 / jnp.maximum(cnt_ref[...], 1.0)
    xa = jnp.concatenate([sum_ref[0], sum_ref[1]], axis=1) * rec
    m = jnp.maximum(
        jnp.dot(jnp.concatenate([xa, xae_ref[...]], axis=1), wm_ref[...],
                preferred_element_type=_f32)
        + bm_ref[...], 0.0)
    o_ref[...] = jnp.maximum(
        jnp.dot(jnp.concatenate([xe_ref[...], m], axis=1), wu_ref[...],
                preferred_element_type=_f32)
        + bu_ref[...], 0.0)


def _tc_pool_body(xe_ref, bat_ref, gs_ref, nb_ref):
    i = pl.program_id(0)
    ohf = (bat_ref[...] == _iota_lanes()).astype(_f32)
    gsp = lax.dot_general(ohf, xe_ref[...], (((0,), (0,)), ((), ())),
                          preferred_element_type=_f32,
                          precision=_HI)
    nbp = jnp.broadcast_to(jnp.sum(ohf, axis=0, keepdims=True), (8, DD))

    @pl.when(i == 0)
    def _():
        gs_ref[...] = gsp
        nb_ref[...] = nbp

    @pl.when(i > 0)
    def _():
        gs_ref[...] = gs_ref[...] + gsp
        nb_ref[...] = nb_ref[...] + nbp


def _tc_final_body(gs_ref, nb_ref, bat_ref, xe_ref,
                   wgg_ref, bgg_ref, wro_ref, bro_ref, o_ref):
    ohf = (bat_ref[...] == _iota_lanes()).astype(_f32)
    gbs = jnp.dot(ohf, gs_ref[...], preferred_element_type=_f32,
                  precision=_HI)
    nbn = jnp.maximum(
        jnp.sum(ohf * nb_ref[0:1, :], axis=1, keepdims=True), 1.0)
    gb = (jnp.dot(gbs / nbn, wgg_ref[...], preferred_element_type=_f32)
          + bgg_ref[...])
    inp = jnp.concatenate([jnp.maximum(gb, 0.0), xe_ref[...]], axis=1)
    o_ref[...] = (
        jnp.dot(inp, wro_ref[...], preferred_element_type=_f32)
        + bro_ref[...])


def _rowspec(shape):
    if shape == (NP, DD):
        return pl.BlockSpec((RB, DD), lambda i: (i, 0))
    if shape == (NP, 1):
        return pl.BlockSpec((RB, 1), lambda i: (i, 0))
    if shape == (NC, NP, DH):
        return pl.BlockSpec((NC, RB, DH), lambda i: (0, i, 0))
    return pl.BlockSpec(shape, lambda i: tuple(0 for _ in shape))


def _tc_call(body, args, out_shapes):
    single = not isinstance(out_shapes[0], tuple)
    oss = (out_shapes,) if single else tuple(out_shapes)
    res = pl.pallas_call(
        body,
        grid=(GRID,),
        in_specs=[_rowspec(a.shape) for a in args],
        out_specs=tuple(_rowspec(s) for s in oss) if not single
        else _rowspec(out_shapes),
        out_shape=tuple(jax.ShapeDtypeStruct(s, _f32) for s in oss)
        if not single else jax.ShapeDtypeStruct(out_shapes, _f32),
    )(*args)
    return res


def kernel(x, edge_index, edge_attr, batch, degree,
           W_en, b_en, W_ne, b_ne, W_ag, b_ag,
           W_m0, b_m0, W_u0, b_u0, W_m1, b_m1, W_u1, b_u1,
           W_m2, b_m2, W_u2, b_u2, W_gg, b_gg, W_ro, b_ro):
    x2 = jnp.pad(x.reshape(NN, DD), ((0, NP - NN), (0, 0)))
    col = edge_index[0]
    row = edge_index[1]
    ea = edge_attr[:, 0]
    npad = EP - EE
    pad_rows = NN + (jnp.arange(npad, dtype=jnp.int32) % (NP - NN))
    col_p = jnp.concatenate(
        [col, jnp.zeros((npad,), jnp.int32)]).reshape(NS, NCHUNK, KCH)
    row_p = jnp.concatenate([row, pad_rows]).reshape(NS, NCHUNK, KCH)
    ea_p = jnp.concatenate(
        [ea, jnp.zeros((npad,), _f32)]).reshape(NS, NCHUNK, KCH)
    bat_p = jnp.concatenate(
        [batch, jnp.full((NP - NN,), PAD_BATCH, jnp.int32)]).reshape(NP, 1)
    deg_p = jnp.pad(degree, (0, NP - NN)).reshape(NP, 1)

    b2 = lambda b: b.reshape(1, -1)
    wro = jnp.pad(W_ro, ((0, 0), (0, DD - 1)))
    bro = jnp.pad(b_ro, (0, DD - 1)).reshape(1, DD)

    x_emb = _tc_call(_tc_embed_body, (x2, W_en, b2(b_en)), (NP, DD))
    degmax = _tc_call(_tc_deg_body, (deg_p, bat_p), (8, DD))

    sc_pass0, sc_wpass = _sc_kernels()
    s0, cnt, eas = sc_pass0(x2.reshape(NP * 2, DH), col_p, row_p, ea_p)
    cnt3 = cnt.reshape(NP, 1)
    eas3 = eas.reshape(NP, 1)
    x_emb, _ = lax.optimization_barrier((x_emb, s0))

    x_agg_emb = _tc_call(
        _tc_stage2_body,
        (s0, cnt3, eas3, bat_p, degmax, W_ne, b2(b_ne), W_ag, b2(b_ag)),
        (NP, DD))

    for Wm, bm, Wu, bu in ((W_m0, b_m0, W_u0, b_u0),
                           (W_m1, b_m1, W_u1, b_u1),
                           (W_m2, b_m2, W_u2, b_u2)):
        sl = sc_wpass(x_emb.reshape(NP * 2, DH), col_p, row_p, ea_p)
        x_emb = _tc_call(
            _tc_layer_body,
            (sl, cnt3, x_emb, x_agg_emb, Wm, b2(bm), Wu, b2(bu)), (NP, DD))

    gs, nb = _tc_call(
        _tc_pool_body, (x_emb, bat_p), ((DD, DD), (8, DD)))
    q = _tc_call(
        _tc_final_body,
        (gs, nb, bat_p, x_emb, W_gg, b2(b_gg), wro, bro), (NP, DD))
    return q[:NN, 0:1]
